# baseline (speedup 1.0000x reference)
import functools

import jax
import jax.numpy as jnp
from jax import lax
from jax.experimental import pallas as pl
from jax.experimental.pallas import tpu as pltpu
from jax.experimental.pallas import tpu_sc as plsc

VOCAB = 1000000
B = 4096
L = 200
N_DOMAIN = 20
N_KELAS = 9
D_OUT = 31
D_PAD = 32


_VB = 2000


def _build_body(dom_ref, kel_ref, reg_ref, ket_ref, out_ref):
    dom = dom_ref[...]
    s = jnp.sum(dom, axis=-1, keepdims=True)
    s_safe = jnp.where(s > 0, s, 1.0)
    out_ref[:, 0:N_DOMAIN] = dom / s_safe
    out_ref[:, N_DOMAIN:N_DOMAIN + N_KELAS] = kel_ref[...]
    out_ref[:, N_DOMAIN + N_KELAS:N_DOMAIN + N_KELAS + 1] = reg_ref[...]
    out_ref[:, N_DOMAIN + N_KELAS + 1:D_OUT] = 1.0 - ket_ref[...]
    out_ref[:, D_OUT:D_PAD] = jnp.zeros((out_ref.shape[0], 1), jnp.float32)


def _build_table(dom, kel, reg, ket):
    grid = VOCAB // _VB
    return pl.pallas_call(
        _build_body,
        grid=(grid,),
        in_specs=[
            pl.BlockSpec((_VB, N_DOMAIN), lambda i: (i, 0)),
            pl.BlockSpec((_VB, N_KELAS), lambda i: (i, 0)),
            pl.BlockSpec((_VB, 1), lambda i: (i, 0)),
            pl.BlockSpec((_VB, 1), lambda i: (i, 0)),
        ],
        out_specs=pl.BlockSpec((_VB, D_PAD), lambda i: (i, 0)),
        out_shape=jax.ShapeDtypeStruct((VOCAB, D_PAD), jnp.float32),
    )(dom, kel, reg, ket)


_NW = 32
_GRP = 128
_G = 8
_CHUNK = _GRP * _G
_R = B * L
_RPW = _R // _NW
_CPW = _RPW // _CHUNK


def _make_gather():
    mesh = plsc.VectorSubcoreMesh(core_axis_name="c", subcore_axis_name="s")

    @functools.partial(
        pl.kernel,
        mesh=mesh,
        out_type=jax.ShapeDtypeStruct((_R, D_PAD), jnp.float32),
        scratch_types=(
            [pltpu.VMEM((_GRP,), jnp.int32) for _ in range(_G)]
            + [pltpu.VMEM((_GRP, D_PAD), jnp.float32) for _ in range(_G)]
            + [pltpu.SemaphoreType.DMA]
        ),
        compiler_params=pltpu.CompilerParams(use_tc_tiling_on_sc=False),
    )
    def gather_k(table_hbm, idx_hbm, out_hbm, *scratch):
        idx_vs = scratch[:_G]
        row_vs = scratch[_G:2 * _G]
        sem = scratch[2 * _G]
        wid = lax.axis_index("s") * 2 + lax.axis_index("c")
        rbase0 = wid * _RPW

        def chunk_body(ci, carry):
            rbase = rbase0 + ci * _CHUNK
            for j in range(_G):
                pltpu.sync_copy(idx_hbm.at[pl.ds(rbase + j * _GRP, _GRP)], idx_vs[j])
            copies = [
                pltpu.async_copy(table_hbm.at[idx_vs[j]], row_vs[j], sem)
                for j in range(_G)
            ]
            for cp in copies:
                cp.wait()
            for j in range(_G):
                pltpu.sync_copy(row_vs[j], out_hbm.at[pl.ds(rbase + j * _GRP, _GRP)])
            return carry

        lax.fori_loop(0, _CPW, chunk_body, 0)

    return gather_k


_gather = _make_gather()


def kernel(domain_table, kelas_table, register, ketidakpastian, indices):
    ft = _build_table(
        domain_table,
        kelas_table,
        register.reshape(VOCAB, 1),
        ketidakpastian.reshape(VOCAB, 1),
    )
    idx_flat = indices.reshape(_R).astype(jnp.int32)
    out_flat = _gather(ft, idx_flat)
    return out_flat[:, :D_OUT].reshape(B, L, D_OUT)


# speedup vs baseline: 3.0330x; 3.0330x over previous
import functools

import jax
import jax.numpy as jnp
from jax import lax
from jax.experimental import pallas as pl
from jax.experimental.pallas import tpu as pltpu
from jax.experimental.pallas import tpu_sc as plsc

VOCAB = 1000000
B = 4096
L = 200
N_DOMAIN = 20
N_KELAS = 9
D_OUT = 31
D_PAD = 32


_VB = 8192


def _build_body(dom_ref, kel_ref, rk_ref, out_ref):
    dom = dom_ref[...]
    s = jnp.sum(dom, axis=0, keepdims=True)
    s_safe = jnp.where(s > 0, s, 1.0)
    scale = jnp.where(s > 0, 1.0 / s_safe, 1.0)
    rk = rk_ref[...]
    t = jnp.concatenate(
        [
            dom * scale,
            kel_ref[...],
            rk[0:1, :],
            1.0 - rk[1:2, :],
            jnp.zeros((1, dom.shape[1]), jnp.float32),
        ],
        axis=0,
    )
    y = t.T
    z = y.reshape(y.shape[0] // 4, 4, 32)
    for f in range(4):
        out_ref[:, f * 32:(f + 1) * 32] = z[:, f, :]


def _build_table(dom_t, kel_t, rk):
    grid = (VOCAB + _VB - 1) // _VB
    return pl.pallas_call(
        _build_body,
        grid=(grid,),
        in_specs=[
            pl.BlockSpec((N_DOMAIN, _VB), lambda i: (0, i)),
            pl.BlockSpec((N_KELAS, _VB), lambda i: (0, i)),
            pl.BlockSpec((2, _VB), lambda i: (0, i)),
        ],
        out_specs=pl.BlockSpec((_VB // 4, 128), lambda i: (i, 0)),
        out_shape=jax.ShapeDtypeStruct((VOCAB * D_PAD // 128, 128), jnp.float32),
    )(dom_t, kel_t, rk)


_NW = 32
_GRP = 128
_G = 8
_CHUNK = _GRP * _G
_R = B * L
_RPW = _R // _NW
_CPW = _RPW // _CHUNK


def _make_gather():
    mesh = plsc.VectorSubcoreMesh(core_axis_name="c", subcore_axis_name="s")

    @functools.partial(
        pl.kernel,
        mesh=mesh,
        out_type=jax.ShapeDtypeStruct((_R, D_PAD), jnp.float32),
        scratch_types=(
            [pltpu.VMEM((_GRP,), jnp.int32) for _ in range(_G)]
            + [pltpu.VMEM((_GRP, D_PAD), jnp.float32) for _ in range(_G)]
            + [pltpu.SemaphoreType.DMA]
        ),
        compiler_params=pltpu.CompilerParams(use_tc_tiling_on_sc=False),
    )
    def gather_k(table_hbm, idx_hbm, out_hbm, *scratch):
        idx_vs = scratch[:_G]
        row_vs = scratch[_G:2 * _G]
        sem = scratch[2 * _G]
        wid = lax.axis_index("s") * 2 + lax.axis_index("c")
        rbase0 = wid * _RPW

        def chunk_body(ci, carry):
            rbase = rbase0 + ci * _CHUNK
            for j in range(_G):
                pltpu.sync_copy(idx_hbm.at[pl.ds(rbase + j * _GRP, _GRP)], idx_vs[j])
            copies = [
                pltpu.async_copy(table_hbm.at[idx_vs[j]], row_vs[j], sem)
                for j in range(_G)
            ]
            for cp in copies:
                cp.wait()
            for j in range(_G):
                pltpu.sync_copy(row_vs[j], out_hbm.at[pl.ds(rbase + j * _GRP, _GRP)])
            return carry

        lax.fori_loop(0, _CPW, chunk_body, 0)

    return gather_k


_gather = _make_gather()


def kernel(domain_table, kelas_table, register, ketidakpastian, indices):
    rk = jnp.stack([register, ketidakpastian])
    ft128 = _build_table(domain_table.T, kelas_table.T, rk)
    ft = ft128.reshape(VOCAB, D_PAD)
    idx_flat = indices.reshape(_R).astype(jnp.int32)
    out_flat = _gather(ft, idx_flat)
    return out_flat[:, :D_OUT].reshape(B, L, D_OUT)


# speedup vs baseline: 3.3815x; 1.1149x over previous
import functools

import jax
import jax.numpy as jnp
from jax import lax
from jax.experimental import pallas as pl
from jax.experimental.pallas import tpu as pltpu
from jax.experimental.pallas import tpu_sc as plsc

VOCAB = 1000000
B = 4096
L = 200
N_DOMAIN = 20
N_KELAS = 9
D_OUT = 31
D_PAD = 32


_VB = 8192


def _build_body(dom_ref, kel_ref, rk_ref, out_ref):
    dom = dom_ref[...]
    s = jnp.sum(dom, axis=0, keepdims=True)
    s_safe = jnp.where(s > 0, s, 1.0)
    scale = jnp.where(s > 0, 1.0 / s_safe, 1.0)
    rk = rk_ref[...]
    t = jnp.concatenate(
        [
            dom * scale,
            kel_ref[...],
            rk[0:1, :],
            1.0 - rk[1:2, :],
            jnp.zeros((1, dom.shape[1]), jnp.float32),
        ],
        axis=0,
    )
    y = t.T
    z = y.reshape(y.shape[0] // 4, 4, 32)
    for f in range(4):
        out_ref[:, f * 32:(f + 1) * 32] = z[:, f, :]


def _build_table(dom_t, kel_t, rk):
    grid = (VOCAB + _VB - 1) // _VB
    return pl.pallas_call(
        _build_body,
        grid=(grid,),
        in_specs=[
            pl.BlockSpec((N_DOMAIN, _VB), lambda i: (0, i)),
            pl.BlockSpec((N_KELAS, _VB), lambda i: (0, i)),
            pl.BlockSpec((2, _VB), lambda i: (0, i)),
        ],
        out_specs=pl.BlockSpec((_VB // 4, 128), lambda i: (i, 0)),
        out_shape=jax.ShapeDtypeStruct((VOCAB * D_PAD // 128, 128), jnp.float32),
    )(dom_t, kel_t, rk)


_NW = 32
_GRP = 128
_G = 10
_CHUNK = _GRP * _G
_R = B * L
_RPW = _R // _NW
_CPW = _RPW // _CHUNK


def _make_gather():
    mesh = plsc.VectorSubcoreMesh(core_axis_name="c", subcore_axis_name="s")

    @functools.partial(
        pl.kernel,
        mesh=mesh,
        out_type=jax.ShapeDtypeStruct((_R, D_PAD), jnp.float32),
        scratch_types=(
            [pltpu.VMEM((_GRP,), jnp.int32) for _ in range(2 * _G)]
            + [pltpu.VMEM((_CHUNK, D_PAD), jnp.float32) for _ in range(2)]
            + [pltpu.SemaphoreType.DMA for _ in range(4)]
        ),
        compiler_params=pltpu.CompilerParams(use_tc_tiling_on_sc=False),
    )
    def gather_k(table_hbm, idx_hbm, out_hbm, *scratch):
        idx_vs = [scratch[:_G], scratch[_G:2 * _G]]
        rows_v = [scratch[2 * _G], scratch[2 * _G + 1]]
        gsem = [scratch[2 * _G + 2], scratch[2 * _G + 3]]
        isem = [scratch[2 * _G + 4], scratch[2 * _G + 5]]
        wid = lax.axis_index("s") * 2 + lax.axis_index("c")
        rbase0 = wid * _RPW
        last = _CPW - 1

        def fire(s, ci):
            rbase = rbase0 + ci * _CHUNK
            icopies = [
                pltpu.async_copy(
                    idx_hbm.at[pl.ds(rbase + j * _GRP, _GRP)], idx_vs[s][j], isem[s]
                )
                for j in range(_G)
            ]
            for cp in icopies:
                cp.wait()
            for j in range(_G):
                pltpu.async_copy(
                    table_hbm.at[idx_vs[s][j]],
                    rows_v[s].at[pl.ds(j * _GRP, _GRP)],
                    gsem[s],
                )

        def drain(s):
            for j in range(_G):
                pltpu.make_async_copy(
                    table_hbm.at[idx_vs[s][j]],
                    rows_v[s].at[pl.ds(j * _GRP, _GRP)],
                    gsem[s],
                ).wait()

        def wb(s, ci):
            rbase = rbase0 + ci * _CHUNK
            pltpu.sync_copy(rows_v[s], out_hbm.at[pl.ds(rbase, _CHUNK)])

        fire(0, 0)

        def pair_body(p, carry):
            ce = 2 * p
            fire(1, ce + 1)
            drain(0)
            wb(0, ce)
            fire(0, jnp.minimum(ce + 2, last))
            drain(1)
            wb(1, ce + 1)
            return carry

        lax.fori_loop(0, _CPW // 2, pair_body, 0)
        drain(0)

    return gather_k


_gather = _make_gather()


def kernel(domain_table, kelas_table, register, ketidakpastian, indices):
    rk = jnp.stack([register, ketidakpastian])
    ft128 = _build_table(domain_table.T, kelas_table.T, rk)
    ft = ft128.reshape(VOCAB, D_PAD)
    idx_flat = indices.reshape(_R).astype(jnp.int32)
    out_flat = _gather(ft, idx_flat)
    return out_flat[:, :D_OUT].reshape(B, L, D_OUT)


# speedup vs baseline: 3.3818x; 1.0001x over previous
import functools

import jax
import jax.numpy as jnp
from jax import lax
from jax.experimental import pallas as pl
from jax.experimental.pallas import tpu as pltpu
from jax.experimental.pallas import tpu_sc as plsc

VOCAB = 1000000
B = 4096
L = 200
N_DOMAIN = 20
N_KELAS = 9
D_OUT = 31
D_PAD = 32


_VB = 8192


def _build_body(dom_ref, kel_ref, rk_ref, out_ref):
    dom = dom_ref[...]
    s = jnp.sum(dom, axis=0, keepdims=True)
    s_safe = jnp.where(s > 0, s, 1.0)
    scale = jnp.where(s > 0, 1.0 / s_safe, 1.0)
    rk = rk_ref[...]
    t = jnp.concatenate(
        [
            dom * scale,
            kel_ref[...],
            rk[0:1, :],
            1.0 - rk[1:2, :],
            jnp.zeros((1, dom.shape[1]), jnp.float32),
        ],
        axis=0,
    )
    y = t.T
    z = y.reshape(y.shape[0] // 4, 4, 32)
    for f in range(4):
        out_ref[:, f * 32:(f + 1) * 32] = z[:, f, :]


def _build_table(dom_t, kel_t, rk):
    grid = (VOCAB + _VB - 1) // _VB
    return pl.pallas_call(
        _build_body,
        grid=(grid,),
        in_specs=[
            pl.BlockSpec((N_DOMAIN, _VB), lambda i: (0, i)),
            pl.BlockSpec((N_KELAS, _VB), lambda i: (0, i)),
            pl.BlockSpec((2, _VB), lambda i: (0, i)),
        ],
        out_specs=pl.BlockSpec((_VB // 4, 128), lambda i: (i, 0)),
        out_shape=jax.ShapeDtypeStruct((VOCAB * D_PAD // 128, 128), jnp.float32),
    )(dom_t, kel_t, rk)


_NW = 32
_GRP = 128
_G = 10
_CHUNK = _GRP * _G
_R = B * L
_RPW = _R // _NW
_CPW = _RPW // _CHUNK


def _make_gather():
    mesh = plsc.VectorSubcoreMesh(core_axis_name="c", subcore_axis_name="s")

    @functools.partial(
        pl.kernel,
        mesh=mesh,
        out_type=jax.ShapeDtypeStruct((_R, D_PAD), jnp.float32),
        scratch_types=(
            [pltpu.VMEM((_GRP,), jnp.int32) for _ in range(2 * _G)]
            + [pltpu.VMEM((_CHUNK, D_PAD), jnp.float32) for _ in range(2)]
            + [pltpu.SemaphoreType.DMA for _ in range(4)]
        ),
        compiler_params=pltpu.CompilerParams(use_tc_tiling_on_sc=False),
    )
    def gather_k(table_hbm, idx_hbm, out_hbm, *scratch):
        idx_vs = [scratch[:_G], scratch[_G:2 * _G]]
        rows_v = [scratch[2 * _G], scratch[2 * _G + 1]]
        gsem = [scratch[2 * _G + 2], scratch[2 * _G + 3]]
        isem = [scratch[2 * _G + 4], scratch[2 * _G + 5]]
        wid = lax.axis_index("s") * 2 + lax.axis_index("c")
        rbase0 = wid * _RPW
        last = _CPW - 1

        def fire(s, ci):
            rbase = rbase0 + ci * _CHUNK
            icopies = [
                pltpu.async_copy(
                    idx_hbm.at[pl.ds(rbase + j * _GRP, _GRP)], idx_vs[s][j], isem[s]
                )
                for j in range(_G)
            ]
            for cp in icopies:
                cp.wait()
            for j in range(_G):
                pltpu.async_copy(
                    table_hbm.at[idx_vs[s][j]],
                    rows_v[s].at[pl.ds(j * _GRP, _GRP)],
                    gsem[s],
                )

        def drain(s):
            for j in range(_G):
                pltpu.make_async_copy(
                    table_hbm.at[idx_vs[s][j]],
                    rows_v[s].at[pl.ds(j * _GRP, _GRP)],
                    gsem[s],
                ).wait()

        def wb(s, ci):
            rbase = rbase0 + ci * _CHUNK
            pltpu.sync_copy(rows_v[s], out_hbm.at[pl.ds(rbase, _CHUNK)])

        fire(0, 0)

        def pair_body(p, carry):
            ce = 2 * p
            fire(1, ce + 1)
            drain(0)
            wb(0, ce)
            fire(0, jnp.minimum(ce + 2, last))
            drain(1)
            wb(1, ce + 1)
            return carry

        lax.fori_loop(0, _CPW // 2, pair_body, 0)
        drain(0)

    return gather_k


_gather = _make_gather()


def kernel(domain_table, kelas_table, register, ketidakpastian, indices):
    rk = jnp.stack([register, ketidakpastian])
    ft128 = _build_table(domain_table.T, kelas_table.T, rk)
    ft = ft128.reshape(VOCAB, D_PAD)
    idx_flat = indices.reshape(_R).astype(jnp.int32)
    out_flat = _gather(ft, idx_flat)
    return out_flat.reshape(B, L, D_PAD)[..., :D_OUT]
